# baseline (speedup 1.0000x reference)
import functools

import numpy as np
import jax
import jax.numpy as jnp
from jax import lax
from jax.experimental import pallas as pl
from jax.experimental.pallas import tpu as pltpu
from jax.experimental.pallas import tpu_sc as plsc

PRE_NMS = 1000
POST_NMS = 100
NMS_TH = 0.7
IMG_W = 320.0
IMG_H = 240.0
BBOX_XFORM_CLIP = float(np.log(1000.0 / 16.0))

H, W = 60, 80
A = 3
NANC = H * W * A
NPADS = 15360
HP, WP = H + 2, W + 2
NPIX = H * WP
NFLAT = 5088
NSEL = 1024
BIGI = np.int32(1 << 30)
NEGF = np.float32(-3.0e38)

_ratios = np.asarray((0.5, 1.0, 2.0), dtype=np.float64)
_hr = np.sqrt(_ratios)
_WS_HALF = (32.0 / _hr / 2.0).astype(np.float32)
_HS_HALF = (32.0 * _hr / 2.0).astype(np.float32)


def _k1_body(xf_ref, w9_ref, cb_ref, wh_ref, bh_ref, out_ref):
    acc = jnp.zeros((NPIX, 256), jnp.float32)
    for di in range(3):
        for dj in range(3):
            t = di * 3 + dj
            off = di * WP + dj
            acc = acc + lax.dot_general(
                xf_ref[0, :, pl.ds(off, NPIX)], w9_ref[t],
                dimension_numbers=(((0,), (0,)), ((), ())),
                preferred_element_type=jnp.float32)
    tact = jnp.maximum(acc + cb_ref[...], 0.0)
    o = jnp.dot(tact, wh_ref[...], preferred_element_type=jnp.float32)
    o = o + bh_ref[...]
    lanei = lax.broadcasted_iota(jnp.int32, (NPIX, 16), 1)
    sig = 1.0 / (1.0 + jnp.exp(-o))
    out_ref[0] = jnp.where(lanei < 3, sig, o)


NSRT = 16384
SR, SC = 128, 128


def _k2_body(s_ref, r_ref, so_ref, fi_ref, tab_ref, m_ref):
    ir = lax.broadcasted_iota(jnp.int32, (SR, SC), 0)
    ic = lax.broadcasted_iota(jnp.int32, (SR, SC), 1)
    i = ir * SC + ic
    idx = i

    a = idx % 3
    hw = idx // 3
    wc = (hw % W).astype(jnp.float32)
    hr = (hw // W).astype(jnp.float32)
    wh0 = jnp.where(a == 0, _WS_HALF[0],
                    jnp.where(a == 1, _WS_HALF[1], _WS_HALF[2]))
    hh0 = jnp.where(a == 0, _HS_HALF[0],
                    jnp.where(a == 1, _HS_HALF[1], _HS_HALF[2]))
    cx0 = wc * 4.0
    cy0 = hr * 4.0
    ax1 = cx0 - wh0
    ax2 = cx0 + wh0
    ay1 = cy0 - hh0
    ay2 = cy0 + hh0
    aw = ax2 - ax1
    ah = ay2 - ay1
    acx = ax1 + 0.5 * aw
    acy = ay1 + 0.5 * ah

    dx = r_ref[0, 0]
    dy = r_ref[0, 1]
    dw = jnp.minimum(r_ref[0, 2], BBOX_XFORM_CLIP)
    dh = jnp.minimum(r_ref[0, 3], BBOX_XFORM_CLIP)
    pcx = dx * aw + acx
    pcy = dy * ah + acy
    pw = jnp.exp(dw) * aw
    ph = jnp.exp(dh) * ah
    bx1 = jnp.clip(pcx - 0.5 * pw, 0.0, IMG_W)
    by1 = jnp.clip(pcy - 0.5 * ph, 0.0, IMG_H)
    bx2 = jnp.clip(pcx + 0.5 * pw, 0.0, IMG_W)
    by2 = jnp.clip(pcy + 0.5 * ph, 0.0, IMG_H)

    s = s_ref[0]
    arrs = [idx, bx1, by1, bx2, by2]
    k = 2
    while k <= NSRT:
        j = k // 2
        while j >= 1:
            if j < SC:
                axis, d = 1, j
                bitclear = (ic & j) == 0
            else:
                axis, d = 0, j // SC
                bitclear = (ir & (j // SC)) == 0
            ps = jnp.where(bitclear, jnp.roll(s, -d, axis=axis),
                           jnp.roll(s, d, axis=axis))
            pi = jnp.where(bitclear, jnp.roll(arrs[0], -d, axis=axis),
                           jnp.roll(arrs[0], d, axis=axis))
            lt = (s > ps) | ((s == ps) & (arrs[0] < pi))
            asc = (i & k) == 0
            sel = (bitclear == asc) == lt
            s = jnp.where(sel, s, ps)
            new_arrs = [jnp.where(sel, arrs[0], pi)]
            for arr in arrs[1:]:
                pa = jnp.where(bitclear, jnp.roll(arr, -d, axis=axis),
                               jnp.roll(arr, d, axis=axis))
                new_arrs.append(jnp.where(sel, arr, pa))
            arrs = new_arrs
            j //= 2
        k *= 2

    lane = lax.broadcasted_iota(jnp.int32, (1, NSEL), 1)
    tsv = s[0:8, :].reshape(1, NSEL)
    x1 = arrs[1][0:8, :].reshape(1, NSEL)
    y1 = arrs[2][0:8, :].reshape(1, NSEL)
    x2 = arrs[3][0:8, :].reshape(1, NSEL)
    y2 = arrs[4][0:8, :].reshape(1, NSEL)

    areas = (x2 - x1) * (y2 - y1)

    x1t = jnp.transpose(x1)
    y1t = jnp.transpose(y1)
    x2t = jnp.transpose(x2)
    y2t = jnp.transpose(y2)
    for g in range(NSEL // 128):
        sl = slice(g * 128, (g + 1) * 128)
        xb1 = x1t[sl, :]
        yb1 = y1t[sl, :]
        xb2 = x2t[sl, :]
        yb2 = y2t[sl, :]
        ab = (xb2 - xb1) * (yb2 - yb1)
        wv = jnp.maximum(jnp.minimum(xb2, x2) - jnp.maximum(xb1, x1), 0.0)
        hv = jnp.maximum(jnp.minimum(yb2, y2) - jnp.maximum(yb1, y1), 0.0)
        inter = wv * hv
        iou = inter / (ab + areas - inter + 1e-9)
        m_ref[sl, :] = (iou > NMS_TH).astype(jnp.float32)

    lane8 = lax.broadcasted_iota(jnp.int32, (1, 8), 1)

    def nms_blk(nb, keep):
        base8 = pl.multiple_of(nb * 8, 8)
        base = nb * 8
        rows = m_ref[pl.ds(base8, 8), :]
        ltm = (lane < base).astype(jnp.float32)
        ext = jnp.max(rows * keep * ltm, axis=1, keepdims=True)
        sub = pltpu.roll(rows, -base, axis=1)[:, 0:8]
        kblk = jnp.ones((1, 8), jnp.float32)
        for q in range(8):
            si = jnp.max(sub[q:q + 1, :] * kblk
                         * (lane8 < q).astype(jnp.float32),
                         axis=1, keepdims=True)
            ki = jnp.where((si > 0.0) | (ext[q:q + 1, :] > 0.0), 0.0, 1.0)
            kblk = jnp.where(lane8 == q, ki, kblk)
        kfull = jnp.pad(kblk, ((0, 0), (0, NSEL - 8)))
        kfull = pltpu.roll(kfull, base, axis=1)
        inblk = (lane >= base) & (lane < base + 8)
        return jnp.where(inblk, kfull, keep)

    keep = lax.fori_loop(0, PRE_NMS // 8, nms_blk,
                         jnp.ones((1, NSEL), jnp.float32))

    fs0 = jnp.where(lane < PRE_NMS,
                    jnp.where(keep > 0.0, tsv, -1.0), NEGF)

    fr = fs0.reshape(8, 128)
    ir8 = lax.broadcasted_iota(jnp.int32, (8, 128), 0)
    ic8 = lax.broadcasted_iota(jnp.int32, (8, 128), 1)
    i8 = ir8 * 128 + ic8
    pos = i8
    k = 2
    while k <= NSEL:
        j = k // 2
        while j >= 1:
            if j < 128:
                axis, d = 1, j
                bitclear = (ic8 & j) == 0
            else:
                axis, d = 0, j // 128
                bitclear = (ir8 & (j // 128)) == 0
            ps = jnp.where(bitclear, jnp.roll(fr, -d, axis=axis),
                           jnp.roll(fr, d, axis=axis))
            pp = jnp.where(bitclear, jnp.roll(pos, -d, axis=axis),
                           jnp.roll(pos, d, axis=axis))
            lt = (fr > ps) | ((fr == ps) & (pos < pp))
            asc = (i8 & k) == 0
            sel = (bitclear == asc) == lt
            fr = jnp.where(sel, fr, ps)
            pos = jnp.where(sel, pos, pp)
            j //= 2
        k *= 2
    so_ref[0] = fr[0:1, :]
    fi_ref[0] = pos[0:1, :]
    tab_ref[0, :, 0:1] = x1t
    tab_ref[0, :, 1:2] = y1t
    tab_ref[0, :, 2:3] = x2t
    tab_ref[0, :, 3:4] = y2t
    tab_ref[0, :, 4:16] = jnp.zeros((NSEL, 12), jnp.float32)


_SC_CORES = 2
_SC_SUBCORES = 16


@functools.lru_cache(maxsize=None)
def _sc_gather(n_rows):
    nw = _SC_CORES * _SC_SUBCORES
    per_w = n_rows // nw
    mesh = plsc.VectorSubcoreMesh(core_axis_name="c", subcore_axis_name="s")

    @functools.partial(
        pl.kernel, mesh=mesh,
        out_type=jax.ShapeDtypeStruct((n_rows, 16), jnp.float32),
        scratch_types=[
            pltpu.VMEM((per_w,), jnp.int32),
            pltpu.VMEM((per_w, 16), jnp.float32),
            pltpu.SemaphoreType.DMA,
        ],
        compiler_params=pltpu.CompilerParams(use_tc_tiling_on_sc=False),
    )
    def gather_k(idx_hbm, table_hbm, out_hbm, idx_v, rows_v, sem):
        wid = lax.axis_index("s") * _SC_CORES + lax.axis_index("c")
        base = wid * per_w
        pltpu.sync_copy(idx_hbm.at[pl.ds(base, per_w)], idx_v)
        pltpu.async_copy(table_hbm.at[idx_v], rows_v, sem).wait()
        pltpu.sync_copy(rows_v, out_hbm.at[pl.ds(base, per_w)])

    return gather_k


@jax.jit
def kernel(features, conv_w, conv_b, cls_w, cls_b, bbox_w, bbox_b):
    B = features.shape[0]
    xp = jnp.pad(features, ((0, 0), (0, 0), (1, 1), (1, 1)))
    xf = xp.reshape(B, 256, HP * WP)
    xf = jnp.pad(xf, ((0, 0), (0, 0), (0, NFLAT - HP * WP)))
    w9 = jnp.stack([conv_w[:, :, i, j].T for i in range(3)
                    for j in range(3)])
    whead = jnp.concatenate(
        [cls_w[:, :, 0, 0], bbox_w[:, :, 0, 0],
         jnp.zeros((1, 256), jnp.float32)], axis=0).T
    bhead = jnp.concatenate(
        [cls_b, bbox_b, jnp.zeros((1,), jnp.float32)]).reshape(1, 16)
    cbr = conv_b.reshape(1, 256)

    out = pl.pallas_call(
        _k1_body,
        grid=(B,),
        in_specs=[
            pl.BlockSpec((1, 256, NFLAT), lambda b: (b, 0, 0)),
            pl.BlockSpec((9, 256, 256), lambda b: (0, 0, 0)),
            pl.BlockSpec((1, 256), lambda b: (0, 0)),
            pl.BlockSpec((256, 16), lambda b: (0, 0)),
            pl.BlockSpec((1, 16), lambda b: (0, 0)),
        ],
        out_specs=pl.BlockSpec((1, NPIX, 16), lambda b: (b, 0, 0)),
        out_shape=jax.ShapeDtypeStruct((B, NPIX, 16), jnp.float32),
    )(xf, w9, cbr, whead, bhead)

    out3 = out.reshape(B, H, WP, 16)[:, :, :W, :]
    scores = out3[..., :3].reshape(B, NANC)
    reg = out3[..., 3:15].reshape(B, NANC, 4)

    scores_p = jnp.pad(scores, ((0, 0), (0, NSRT - NANC)),
                       constant_values=-1.0).reshape(B, SR, SC)
    regp = jnp.pad(reg.transpose(0, 2, 1),
                   ((0, 0), (0, 0), (0, NSRT - NANC)))
    regp = regp.reshape(B, 4, SR, SC)

    scoresp, fi, tab = pl.pallas_call(
        _k2_body,
        grid=(B,),
        in_specs=[
            pl.BlockSpec((1, SR, SC), lambda b: (b, 0, 0)),
            pl.BlockSpec((1, 4, SR, SC), lambda b: (b, 0, 0, 0)),
        ],
        out_specs=[
            pl.BlockSpec((1, 1, 128), lambda b: (b, 0, 0)),
            pl.BlockSpec((1, 1, 128), lambda b: (b, 0, 0)),
            pl.BlockSpec((1, NSEL, 16), lambda b: (b, 0, 0)),
        ],
        out_shape=[
            jax.ShapeDtypeStruct((B, 1, 128), jnp.float32),
            jax.ShapeDtypeStruct((B, 1, 128), jnp.int32),
            jax.ShapeDtypeStruct((B, NSEL, 16), jnp.float32),
        ],
        scratch_shapes=[pltpu.VMEM((NSEL, NSEL), jnp.float32)],
    )(scores_p, regp)

    fig = (fi[:, 0, :] +
           (jnp.arange(B, dtype=jnp.int32) * NSEL)[:, None]).reshape(-1)
    rows2 = _sc_gather(B * 128)(fig, tab.reshape(B * NSEL, 16))
    boxes = rows2.reshape(B, 128, 16)[:, :POST_NMS, 0:4]
    out_scores = scoresp[:, 0, :POST_NMS]
    return boxes, out_scores


# speedup vs baseline: 1.0043x; 1.0043x over previous
import functools

import numpy as np
import jax
import jax.numpy as jnp
from jax import lax
from jax.experimental import pallas as pl
from jax.experimental.pallas import tpu as pltpu
from jax.experimental.pallas import tpu_sc as plsc

PRE_NMS = 1000
POST_NMS = 100
NMS_TH = 0.7
IMG_W = 320.0
IMG_H = 240.0
BBOX_XFORM_CLIP = float(np.log(1000.0 / 16.0))

H, W = 60, 80
A = 3
NANC = H * W * A
NPADS = 15360
HP, WP = H + 2, W + 2
NPIX = H * WP
NFLAT = 5088
NSEL = 1024
BIGI = np.int32(1 << 30)
NEGF = np.float32(-3.0e38)

_ratios = np.asarray((0.5, 1.0, 2.0), dtype=np.float64)
_hr = np.sqrt(_ratios)
_WS_HALF = (32.0 / _hr / 2.0).astype(np.float32)
_HS_HALF = (32.0 * _hr / 2.0).astype(np.float32)


def _k1_body(xf_ref, w9_ref, cb_ref, wh_ref, bh_ref, out_ref):
    acc = jnp.zeros((NPIX, 256), jnp.float32)
    for di in range(3):
        for dj in range(3):
            t = di * 3 + dj
            off = di * WP + dj
            acc = acc + lax.dot_general(
                xf_ref[0, :, pl.ds(off, NPIX)], w9_ref[t],
                dimension_numbers=(((0,), (0,)), ((), ())),
                preferred_element_type=jnp.float32)
    tact = jnp.maximum(acc + cb_ref[...], 0.0)
    o = jnp.dot(tact, wh_ref[...], preferred_element_type=jnp.float32)
    o = o + bh_ref[...]
    lanei = lax.broadcasted_iota(jnp.int32, (NPIX, 16), 1)
    sig = 1.0 / (1.0 + jnp.exp(-o))
    out_ref[0] = jnp.where(lanei < 3, sig, o)


NSRT = 16384
SR, SC = 128, 128


def _k2a_body(s_ref, ts_ref, ix_ref):
    s = s_ref[0]
    ir = lax.broadcasted_iota(jnp.int32, (SR, SC), 0)
    ic = lax.broadcasted_iota(jnp.int32, (SR, SC), 1)
    i = ir * SC + ic
    idx = i

    k = 2
    while k <= NSRT:
        j = k // 2
        while j >= 1:
            if j < SC:
                axis, d = 1, j
                bitclear = (ic & j) == 0
            else:
                axis, d = 0, j // SC
                bitclear = (ir & (j // SC)) == 0
            ps = jnp.where(bitclear, jnp.roll(s, -d, axis=axis),
                           jnp.roll(s, d, axis=axis))
            pi = jnp.where(bitclear, jnp.roll(idx, -d, axis=axis),
                           jnp.roll(idx, d, axis=axis))
            lt = (s > ps) | ((s == ps) & (idx < pi))
            asc = (i & k) == 0
            sel = (bitclear == asc) == lt
            s = jnp.where(sel, s, ps)
            idx = jnp.where(sel, idx, pi)
            j //= 2
        k *= 2
    ts_ref[0] = s[0:8, :].reshape(1, NSEL)
    ix_ref[0] = idx[0:8, :].reshape(1, NSEL)


def _k2b_body(ts_ref, ix_ref, g_ref, so_ref, fi_ref, tab_ref, m_ref):
    idx = ix_ref[0, 0:1, :]
    lane = lax.broadcasted_iota(jnp.int32, (1, NSEL), 1)

    a = idx % 3
    hw = idx // 3
    wc = (hw % W).astype(jnp.float32)
    hr = (hw // W).astype(jnp.float32)
    wh0 = jnp.where(a == 0, _WS_HALF[0],
                    jnp.where(a == 1, _WS_HALF[1], _WS_HALF[2]))
    hh0 = jnp.where(a == 0, _HS_HALF[0],
                    jnp.where(a == 1, _HS_HALF[1], _HS_HALF[2]))
    cx0 = wc * 4.0
    cy0 = hr * 4.0
    ax1 = cx0 - wh0
    ax2 = cx0 + wh0
    ay1 = cy0 - hh0
    ay2 = cy0 + hh0
    aw = ax2 - ax1
    ah = ay2 - ay1
    acx = ax1 + 0.5 * aw
    acy = ay1 + 0.5 * ah

    dx = g_ref[0, 0:1, :]
    dy = g_ref[0, 1:2, :]
    dw = jnp.minimum(g_ref[0, 2:3, :], BBOX_XFORM_CLIP)
    dh = jnp.minimum(g_ref[0, 3:4, :], BBOX_XFORM_CLIP)
    pcx = dx * aw + acx
    pcy = dy * ah + acy
    pw = jnp.exp(dw) * aw
    ph = jnp.exp(dh) * ah
    x1 = jnp.clip(pcx - 0.5 * pw, 0.0, IMG_W)
    y1 = jnp.clip(pcy - 0.5 * ph, 0.0, IMG_H)
    x2 = jnp.clip(pcx + 0.5 * pw, 0.0, IMG_W)
    y2 = jnp.clip(pcy + 0.5 * ph, 0.0, IMG_H)

    areas = (x2 - x1) * (y2 - y1)

    def _ext(vec, ohf):
        return jnp.sum(jnp.sum(ohf * vec, axis=1, keepdims=True), axis=0,
                       keepdims=True)

    x1t = jnp.transpose(x1)
    y1t = jnp.transpose(y1)
    x2t = jnp.transpose(x2)
    y2t = jnp.transpose(y2)
    for g in range(NSEL // 128):
        sl = slice(g * 128, (g + 1) * 128)
        xb1 = x1t[sl, :]
        yb1 = y1t[sl, :]
        xb2 = x2t[sl, :]
        yb2 = y2t[sl, :]
        ab = (xb2 - xb1) * (yb2 - yb1)
        wv = jnp.maximum(jnp.minimum(xb2, x2) - jnp.maximum(xb1, x1), 0.0)
        hv = jnp.maximum(jnp.minimum(yb2, y2) - jnp.maximum(yb1, y1), 0.0)
        inter = wv * hv
        iou = inter / (ab + areas - inter + 1e-9)
        m_ref[sl, :] = (iou > NMS_TH).astype(jnp.float32)

    lane8 = lax.broadcasted_iota(jnp.int32, (1, 8), 1)

    def nms_blk(nb, keep):
        base8 = pl.multiple_of(nb * 8, 8)
        base = nb * 8
        rows = m_ref[pl.ds(base8, 8), :]
        ltm = (lane < base).astype(jnp.float32)
        ext = jnp.max(rows * keep * ltm, axis=1, keepdims=True)
        sub = pltpu.roll(rows, -base, axis=1)[:, 0:8]
        kblk = jnp.ones((1, 8), jnp.float32)
        for q in range(8):
            si = jnp.max(sub[q:q + 1, :] * kblk
                         * (lane8 < q).astype(jnp.float32),
                         axis=1, keepdims=True)
            ki = jnp.where((si > 0.0) | (ext[q:q + 1, :] > 0.0), 0.0, 1.0)
            kblk = jnp.where(lane8 == q, ki, kblk)
        kfull = jnp.pad(kblk, ((0, 0), (0, NSEL - 8)))
        kfull = pltpu.roll(kfull, base, axis=1)
        inblk = (lane >= base) & (lane < base + 8)
        return jnp.where(inblk, kfull, keep)

    keep = lax.fori_loop(0, PRE_NMS // 8, nms_blk,
                         jnp.ones((1, NSEL), jnp.float32))

    tsv = ts_ref[0, 0:1, :]
    fs0 = jnp.where(lane < PRE_NMS,
                    jnp.where(keep > 0.0, tsv, -1.0), NEGF)

    fr = fs0.reshape(8, 128)
    ir8 = lax.broadcasted_iota(jnp.int32, (8, 128), 0)
    ic8 = lax.broadcasted_iota(jnp.int32, (8, 128), 1)
    i8 = ir8 * 128 + ic8
    pos = i8
    k = 2
    while k <= NSEL:
        j = k // 2
        while j >= 1:
            if j < 128:
                axis, d = 1, j
                bitclear = (ic8 & j) == 0
            else:
                axis, d = 0, j // 128
                bitclear = (ir8 & (j // 128)) == 0
            ps = jnp.where(bitclear, jnp.roll(fr, -d, axis=axis),
                           jnp.roll(fr, d, axis=axis))
            pp = jnp.where(bitclear, jnp.roll(pos, -d, axis=axis),
                           jnp.roll(pos, d, axis=axis))
            lt = (fr > ps) | ((fr == ps) & (pos < pp))
            asc = (i8 & k) == 0
            sel = (bitclear == asc) == lt
            fr = jnp.where(sel, fr, ps)
            pos = jnp.where(sel, pos, pp)
            j //= 2
        k *= 2
    so_ref[0] = fr[0:1, :]
    fi_ref[0] = pos[0:1, :]
    tab_ref[0, :, 0:1] = x1t
    tab_ref[0, :, 1:2] = y1t
    tab_ref[0, :, 2:3] = x2t
    tab_ref[0, :, 3:4] = y2t
    tab_ref[0, :, 4:16] = jnp.zeros((NSEL, 12), jnp.float32)


_SC_CORES = 2
_SC_SUBCORES = 16


@functools.lru_cache(maxsize=None)
def _sc_gather(n_rows):
    nw = _SC_CORES * _SC_SUBCORES
    per_w = n_rows // nw
    mesh = plsc.VectorSubcoreMesh(core_axis_name="c", subcore_axis_name="s")

    @functools.partial(
        pl.kernel, mesh=mesh,
        out_type=jax.ShapeDtypeStruct((n_rows, 16), jnp.float32),
        scratch_types=[
            pltpu.VMEM((per_w,), jnp.int32),
            pltpu.VMEM((per_w, 16), jnp.float32),
            pltpu.SemaphoreType.DMA,
        ],
        compiler_params=pltpu.CompilerParams(use_tc_tiling_on_sc=False),
    )
    def gather_k(idx_hbm, table_hbm, out_hbm, idx_v, rows_v, sem):
        wid = lax.axis_index("s") * _SC_CORES + lax.axis_index("c")
        base = wid * per_w
        pltpu.sync_copy(idx_hbm.at[pl.ds(base, per_w)], idx_v)
        pltpu.async_copy(table_hbm.at[idx_v], rows_v, sem).wait()
        pltpu.sync_copy(rows_v, out_hbm.at[pl.ds(base, per_w)])

    return gather_k


@jax.jit
def kernel(features, conv_w, conv_b, cls_w, cls_b, bbox_w, bbox_b):
    B = features.shape[0]
    xp = jnp.pad(features, ((0, 0), (0, 0), (1, 1), (1, 1)))
    xf = xp.reshape(B, 256, HP * WP)
    xf = jnp.pad(xf, ((0, 0), (0, 0), (0, NFLAT - HP * WP)))
    w9 = jnp.stack([conv_w[:, :, i, j].T for i in range(3)
                    for j in range(3)])
    whead = jnp.concatenate(
        [cls_w[:, :, 0, 0], bbox_w[:, :, 0, 0],
         jnp.zeros((1, 256), jnp.float32)], axis=0).T
    bhead = jnp.concatenate(
        [cls_b, bbox_b, jnp.zeros((1,), jnp.float32)]).reshape(1, 16)
    cbr = conv_b.reshape(1, 256)

    out = pl.pallas_call(
        _k1_body,
        grid=(B,),
        in_specs=[
            pl.BlockSpec((1, 256, NFLAT), lambda b: (b, 0, 0)),
            pl.BlockSpec((9, 256, 256), lambda b: (0, 0, 0)),
            pl.BlockSpec((1, 256), lambda b: (0, 0)),
            pl.BlockSpec((256, 16), lambda b: (0, 0)),
            pl.BlockSpec((1, 16), lambda b: (0, 0)),
        ],
        out_specs=pl.BlockSpec((1, NPIX, 16), lambda b: (b, 0, 0)),
        out_shape=jax.ShapeDtypeStruct((B, NPIX, 16), jnp.float32),
    )(xf, w9, cbr, whead, bhead)

    out3 = out.reshape(B, H, WP, 16)[:, :, :W, :]
    scores = out3[..., :3].reshape(B, NANC)
    reg = out3[..., 3:15].reshape(B, NANC, 4)

    scores_p = jnp.pad(scores, ((0, 0), (0, NSRT - NANC)),
                       constant_values=-1.0).reshape(B, SR, SC)

    ts, ix = pl.pallas_call(
        _k2a_body,
        grid=(B,),
        in_specs=[
            pl.BlockSpec((1, SR, SC), lambda b: (b, 0, 0)),
        ],
        out_specs=[
            pl.BlockSpec((1, 1, NSEL), lambda b: (b, 0, 0)),
            pl.BlockSpec((1, 1, NSEL), lambda b: (b, 0, 0)),
        ],
        out_shape=[
            jax.ShapeDtypeStruct((B, 1, NSEL), jnp.float32),
            jax.ShapeDtypeStruct((B, 1, NSEL), jnp.int32),
        ],
    )(scores_p)

    table = jnp.pad(reg, ((0, 0), (0, 0), (0, 12))).reshape(B * NANC, 16)
    idxg = (ix[:, 0, :] +
            (jnp.arange(B, dtype=jnp.int32) * NANC)[:, None]).reshape(-1)
    rows = _sc_gather(B * NSEL)(idxg, table)
    gath = rows.reshape(B, NSEL, 16)[:, :, 0:4].transpose(0, 2, 1)

    scoresp, fi, tab = pl.pallas_call(
        _k2b_body,
        grid=(B,),
        in_specs=[
            pl.BlockSpec((1, 1, NSEL), lambda b: (b, 0, 0)),
            pl.BlockSpec((1, 1, NSEL), lambda b: (b, 0, 0)),
            pl.BlockSpec((1, 4, NSEL), lambda b: (b, 0, 0)),
        ],
        out_specs=[
            pl.BlockSpec((1, 1, 128), lambda b: (b, 0, 0)),
            pl.BlockSpec((1, 1, 128), lambda b: (b, 0, 0)),
            pl.BlockSpec((1, NSEL, 16), lambda b: (b, 0, 0)),
        ],
        out_shape=[
            jax.ShapeDtypeStruct((B, 1, 128), jnp.float32),
            jax.ShapeDtypeStruct((B, 1, 128), jnp.int32),
            jax.ShapeDtypeStruct((B, NSEL, 16), jnp.float32),
        ],
        scratch_shapes=[pltpu.VMEM((NSEL, NSEL), jnp.float32)],
    )(ts, ix, gath)

    fig = (fi[:, 0, :] +
           (jnp.arange(B, dtype=jnp.int32) * NSEL)[:, None]).reshape(-1)
    rows2 = _sc_gather(B * 128)(fig, tab.reshape(B * NSEL, 16))
    boxes = rows2.reshape(B, 128, 16)[:, :POST_NMS, 0:4]
    out_scores = scoresp[:, 0, :POST_NMS]
    return boxes, out_scores


# speedup vs baseline: 1.0511x; 1.0465x over previous
import functools

import numpy as np
import jax
import jax.numpy as jnp
from jax import lax
from jax.experimental import pallas as pl
from jax.experimental.pallas import tpu as pltpu
from jax.experimental.pallas import tpu_sc as plsc

PRE_NMS = 1000
POST_NMS = 100
NMS_TH = 0.7
IMG_W = 320.0
IMG_H = 240.0
BBOX_XFORM_CLIP = float(np.log(1000.0 / 16.0))

H, W = 60, 80
A = 3
NANC = H * W * A
NPADS = 15360
HP, WP = H + 2, W + 2
NPIX = H * WP
NFLAT = 5088
NSEL = 1024
BIGI = np.int32(1 << 30)
NEGF = np.float32(-3.0e38)

_ratios = np.asarray((0.5, 1.0, 2.0), dtype=np.float64)
_hr = np.sqrt(_ratios)
_WS_HALF = (32.0 / _hr / 2.0).astype(np.float32)
_HS_HALF = (32.0 * _hr / 2.0).astype(np.float32)


def _k1_body(xf_ref, w9_ref, cb_ref, wh_ref, bh_ref, out_ref):
    acc = jnp.zeros((NPIX, 256), jnp.float32)
    for di in range(3):
        for dj in range(3):
            t = di * 3 + dj
            off = di * WP + dj
            acc = acc + jnp.dot(
                xf_ref[0, pl.ds(off, NPIX), :], w9_ref[t],
                preferred_element_type=jnp.float32)
    tact = jnp.maximum(acc + cb_ref[...], 0.0)
    o = jnp.dot(tact, wh_ref[...], preferred_element_type=jnp.float32)
    o = o + bh_ref[...]
    lanei = lax.broadcasted_iota(jnp.int32, (NPIX, 16), 1)
    sig = 1.0 / (1.0 + jnp.exp(-o))
    out_ref[0] = jnp.where(lanei < 3, sig, o)


NSRT = 16384
SR, SC = 128, 128


def _k2a_body(s_ref, ts_ref, ix_ref):
    s = s_ref[0]
    ir = lax.broadcasted_iota(jnp.int32, (SR, SC), 0)
    ic = lax.broadcasted_iota(jnp.int32, (SR, SC), 1)
    i = ir * SC + ic
    idx = i

    k = 2
    while k <= NSRT:
        j = k // 2
        while j >= 1:
            if j < SC:
                axis, d = 1, j
                bitclear = (ic & j) == 0
            else:
                axis, d = 0, j // SC
                bitclear = (ir & (j // SC)) == 0
            ps = jnp.where(bitclear, jnp.roll(s, -d, axis=axis),
                           jnp.roll(s, d, axis=axis))
            pi = jnp.where(bitclear, jnp.roll(idx, -d, axis=axis),
                           jnp.roll(idx, d, axis=axis))
            lt = (s > ps) | ((s == ps) & (idx < pi))
            asc = (i & k) == 0
            sel = (bitclear == asc) == lt
            s = jnp.where(sel, s, ps)
            idx = jnp.where(sel, idx, pi)
            j //= 2
        k *= 2
    ts_ref[0] = s[0:8, :].reshape(1, NSEL)
    ix_ref[0] = idx[0:8, :].reshape(1, NSEL)


def _k2b_body(ts_ref, ix_ref, g_ref, so_ref, fi_ref, tab_ref, m_ref):
    idx = ix_ref[0, 0:1, :]
    lane = lax.broadcasted_iota(jnp.int32, (1, NSEL), 1)

    a = idx % 3
    hw = idx // 3
    wc = (hw % W).astype(jnp.float32)
    hr = (hw // W).astype(jnp.float32)
    wh0 = jnp.where(a == 0, _WS_HALF[0],
                    jnp.where(a == 1, _WS_HALF[1], _WS_HALF[2]))
    hh0 = jnp.where(a == 0, _HS_HALF[0],
                    jnp.where(a == 1, _HS_HALF[1], _HS_HALF[2]))
    cx0 = wc * 4.0
    cy0 = hr * 4.0
    ax1 = cx0 - wh0
    ax2 = cx0 + wh0
    ay1 = cy0 - hh0
    ay2 = cy0 + hh0
    aw = ax2 - ax1
    ah = ay2 - ay1
    acx = ax1 + 0.5 * aw
    acy = ay1 + 0.5 * ah

    dx = g_ref[0, 0:1, :]
    dy = g_ref[0, 1:2, :]
    dw = jnp.minimum(g_ref[0, 2:3, :], BBOX_XFORM_CLIP)
    dh = jnp.minimum(g_ref[0, 3:4, :], BBOX_XFORM_CLIP)
    pcx = dx * aw + acx
    pcy = dy * ah + acy
    pw = jnp.exp(dw) * aw
    ph = jnp.exp(dh) * ah
    x1 = jnp.clip(pcx - 0.5 * pw, 0.0, IMG_W)
    y1 = jnp.clip(pcy - 0.5 * ph, 0.0, IMG_H)
    x2 = jnp.clip(pcx + 0.5 * pw, 0.0, IMG_W)
    y2 = jnp.clip(pcy + 0.5 * ph, 0.0, IMG_H)

    areas = (x2 - x1) * (y2 - y1)

    def _ext(vec, ohf):
        return jnp.sum(jnp.sum(ohf * vec, axis=1, keepdims=True), axis=0,
                       keepdims=True)

    x1t = jnp.transpose(x1)
    y1t = jnp.transpose(y1)
    x2t = jnp.transpose(x2)
    y2t = jnp.transpose(y2)
    for g in range(NSEL // 128):
        sl = slice(g * 128, (g + 1) * 128)
        xb1 = x1t[sl, :]
        yb1 = y1t[sl, :]
        xb2 = x2t[sl, :]
        yb2 = y2t[sl, :]
        ab = (xb2 - xb1) * (yb2 - yb1)
        wv = jnp.maximum(jnp.minimum(xb2, x2) - jnp.maximum(xb1, x1), 0.0)
        hv = jnp.maximum(jnp.minimum(yb2, y2) - jnp.maximum(yb1, y1), 0.0)
        inter = wv * hv
        iou = inter / (ab + areas - inter + 1e-9)
        m_ref[sl, :] = (iou > NMS_TH).astype(jnp.float32)

    lane8 = lax.broadcasted_iota(jnp.int32, (1, 8), 1)

    def nms_blk(nb, keep):
        base8 = pl.multiple_of(nb * 8, 8)
        base = nb * 8
        rows = m_ref[pl.ds(base8, 8), :]
        ltm = (lane < base).astype(jnp.float32)
        ext = jnp.max(rows * keep * ltm, axis=1, keepdims=True)
        sub = pltpu.roll(rows, -base, axis=1)[:, 0:8]
        kblk = jnp.ones((1, 8), jnp.float32)
        for q in range(8):
            si = jnp.max(sub[q:q + 1, :] * kblk
                         * (lane8 < q).astype(jnp.float32),
                         axis=1, keepdims=True)
            ki = jnp.where((si > 0.0) | (ext[q:q + 1, :] > 0.0), 0.0, 1.0)
            kblk = jnp.where(lane8 == q, ki, kblk)
        kfull = jnp.pad(kblk, ((0, 0), (0, NSEL - 8)))
        kfull = pltpu.roll(kfull, base, axis=1)
        inblk = (lane >= base) & (lane < base + 8)
        return jnp.where(inblk, kfull, keep)

    keep = lax.fori_loop(0, PRE_NMS // 8, nms_blk,
                         jnp.ones((1, NSEL), jnp.float32))

    tsv = ts_ref[0, 0:1, :]
    fs0 = jnp.where(lane < PRE_NMS,
                    jnp.where(keep > 0.0, tsv, -1.0), NEGF)

    fr = fs0.reshape(8, 128)
    ir8 = lax.broadcasted_iota(jnp.int32, (8, 128), 0)
    ic8 = lax.broadcasted_iota(jnp.int32, (8, 128), 1)
    i8 = ir8 * 128 + ic8
    pos = i8
    k = 2
    while k <= NSEL:
        j = k // 2
        while j >= 1:
            if j < 128:
                axis, d = 1, j
                bitclear = (ic8 & j) == 0
            else:
                axis, d = 0, j // 128
                bitclear = (ir8 & (j // 128)) == 0
            ps = jnp.where(bitclear, jnp.roll(fr, -d, axis=axis),
                           jnp.roll(fr, d, axis=axis))
            pp = jnp.where(bitclear, jnp.roll(pos, -d, axis=axis),
                           jnp.roll(pos, d, axis=axis))
            lt = (fr > ps) | ((fr == ps) & (pos < pp))
            asc = (i8 & k) == 0
            sel = (bitclear == asc) == lt
            fr = jnp.where(sel, fr, ps)
            pos = jnp.where(sel, pos, pp)
            j //= 2
        k *= 2
    so_ref[0] = fr[0:1, :]
    fi_ref[0] = pos[0:1, :]
    tab_ref[0, :, 0:1] = x1t
    tab_ref[0, :, 1:2] = y1t
    tab_ref[0, :, 2:3] = x2t
    tab_ref[0, :, 3:4] = y2t
    tab_ref[0, :, 4:16] = jnp.zeros((NSEL, 12), jnp.float32)


_SC_CORES = 2
_SC_SUBCORES = 16


@functools.lru_cache(maxsize=None)
def _sc_gather(n_rows):
    nw = _SC_CORES * _SC_SUBCORES
    per_w = n_rows // nw
    mesh = plsc.VectorSubcoreMesh(core_axis_name="c", subcore_axis_name="s")

    @functools.partial(
        pl.kernel, mesh=mesh,
        out_type=jax.ShapeDtypeStruct((n_rows, 16), jnp.float32),
        scratch_types=[
            pltpu.VMEM((per_w,), jnp.int32),
            pltpu.VMEM((per_w, 16), jnp.float32),
            pltpu.SemaphoreType.DMA,
        ],
        compiler_params=pltpu.CompilerParams(use_tc_tiling_on_sc=False),
    )
    def gather_k(idx_hbm, table_hbm, out_hbm, idx_v, rows_v, sem):
        wid = lax.axis_index("s") * _SC_CORES + lax.axis_index("c")
        base = wid * per_w
        pltpu.sync_copy(idx_hbm.at[pl.ds(base, per_w)], idx_v)
        pltpu.async_copy(table_hbm.at[idx_v], rows_v, sem).wait()
        pltpu.sync_copy(rows_v, out_hbm.at[pl.ds(base, per_w)])

    return gather_k


@jax.jit
def kernel(features, conv_w, conv_b, cls_w, cls_b, bbox_w, bbox_b):
    B = features.shape[0]
    xp = jnp.pad(features, ((0, 0), (0, 0), (1, 1), (1, 1)))
    xfT = xp.reshape(B, 256, HP * WP).transpose(0, 2, 1)
    xfT = jnp.pad(xfT, ((0, 0), (0, NFLAT - HP * WP), (0, 0)))
    w9 = jnp.stack([conv_w[:, :, i, j].T for i in range(3)
                    for j in range(3)])
    whead = jnp.concatenate(
        [cls_w[:, :, 0, 0], bbox_w[:, :, 0, 0],
         jnp.zeros((1, 256), jnp.float32)], axis=0).T
    bhead = jnp.concatenate(
        [cls_b, bbox_b, jnp.zeros((1,), jnp.float32)]).reshape(1, 16)
    cbr = conv_b.reshape(1, 256)

    out = pl.pallas_call(
        _k1_body,
        grid=(B,),
        in_specs=[
            pl.BlockSpec((1, NFLAT, 256), lambda b: (b, 0, 0)),
            pl.BlockSpec((9, 256, 256), lambda b: (0, 0, 0)),
            pl.BlockSpec((1, 256), lambda b: (0, 0)),
            pl.BlockSpec((256, 16), lambda b: (0, 0)),
            pl.BlockSpec((1, 16), lambda b: (0, 0)),
        ],
        out_specs=pl.BlockSpec((1, NPIX, 16), lambda b: (b, 0, 0)),
        out_shape=jax.ShapeDtypeStruct((B, NPIX, 16), jnp.float32),
    )(xfT, w9, cbr, whead, bhead)

    out3 = out.reshape(B, H, WP, 16)[:, :, :W, :]
    scores = out3[..., :3].reshape(B, NANC)
    reg = out3[..., 3:15].reshape(B, NANC, 4)

    scores_p = jnp.pad(scores, ((0, 0), (0, NSRT - NANC)),
                       constant_values=-1.0).reshape(B, SR, SC)

    ts, ix = pl.pallas_call(
        _k2a_body,
        grid=(B,),
        in_specs=[
            pl.BlockSpec((1, SR, SC), lambda b: (b, 0, 0)),
        ],
        out_specs=[
            pl.BlockSpec((1, 1, NSEL), lambda b: (b, 0, 0)),
            pl.BlockSpec((1, 1, NSEL), lambda b: (b, 0, 0)),
        ],
        out_shape=[
            jax.ShapeDtypeStruct((B, 1, NSEL), jnp.float32),
            jax.ShapeDtypeStruct((B, 1, NSEL), jnp.int32),
        ],
    )(scores_p)

    table = jnp.pad(reg, ((0, 0), (0, 0), (0, 12))).reshape(B * NANC, 16)
    idxg = (ix[:, 0, :] +
            (jnp.arange(B, dtype=jnp.int32) * NANC)[:, None]).reshape(-1)
    rows = _sc_gather(B * NSEL)(idxg, table)
    gath = rows.reshape(B, NSEL, 16)[:, :, 0:4].transpose(0, 2, 1)

    scoresp, fi, tab = pl.pallas_call(
        _k2b_body,
        grid=(B,),
        in_specs=[
            pl.BlockSpec((1, 1, NSEL), lambda b: (b, 0, 0)),
            pl.BlockSpec((1, 1, NSEL), lambda b: (b, 0, 0)),
            pl.BlockSpec((1, 4, NSEL), lambda b: (b, 0, 0)),
        ],
        out_specs=[
            pl.BlockSpec((1, 1, 128), lambda b: (b, 0, 0)),
            pl.BlockSpec((1, 1, 128), lambda b: (b, 0, 0)),
            pl.BlockSpec((1, NSEL, 16), lambda b: (b, 0, 0)),
        ],
        out_shape=[
            jax.ShapeDtypeStruct((B, 1, 128), jnp.float32),
            jax.ShapeDtypeStruct((B, 1, 128), jnp.int32),
            jax.ShapeDtypeStruct((B, NSEL, 16), jnp.float32),
        ],
        scratch_shapes=[pltpu.VMEM((NSEL, NSEL), jnp.float32)],
    )(ts, ix, gath)

    fig = (fi[:, 0, :] +
           (jnp.arange(B, dtype=jnp.int32) * NSEL)[:, None]).reshape(-1)
    rows2 = _sc_gather(B * 128)(fig, tab.reshape(B * NSEL, 16))
    boxes = rows2.reshape(B, 128, 16)[:, :POST_NMS, 0:4]
    out_scores = scoresp[:, 0, :POST_NMS]
    return boxes, out_scores


# speedup vs baseline: 1.0749x; 1.0227x over previous
import functools

import numpy as np
import jax
import jax.numpy as jnp
from jax import lax
from jax.experimental import pallas as pl
from jax.experimental.pallas import tpu as pltpu
from jax.experimental.pallas import tpu_sc as plsc

PRE_NMS = 1000
POST_NMS = 100
NMS_TH = 0.7
IMG_W = 320.0
IMG_H = 240.0
BBOX_XFORM_CLIP = float(np.log(1000.0 / 16.0))

H, W = 60, 80
A = 3
NANC = H * W * A
NPADS = 15360
HP, WP = H + 2, W + 2
NPIX = H * WP
NFLAT = 5088
NSEL = 1024
BIGI = np.int32(1 << 30)
NEGF = np.float32(-3.0e38)

_ratios = np.asarray((0.5, 1.0, 2.0), dtype=np.float64)
_hr = np.sqrt(_ratios)
_WS_HALF = (32.0 / _hr / 2.0).astype(np.float32)
_HS_HALF = (32.0 * _hr / 2.0).astype(np.float32)


def _k1_body(xf_ref, w9_ref, cb_ref, wh_ref, bh_ref, out_ref):
    acc = jnp.zeros((NPIX, 256), jnp.float32)
    for di in range(3):
        for dj in range(3):
            t = di * 3 + dj
            off = di * WP + dj
            acc = acc + lax.dot_general(
                xf_ref[0, pl.ds(off, NPIX), :], w9_ref[t],
                dimension_numbers=(((1,), (1,)), ((), ())),
                preferred_element_type=jnp.float32)
    tact = jnp.maximum(acc + cb_ref[...], 0.0)
    o = jnp.dot(tact, wh_ref[...], preferred_element_type=jnp.float32)
    o = o + bh_ref[...]
    lanei = lax.broadcasted_iota(jnp.int32, (NPIX, 16), 1)
    sig = 1.0 / (1.0 + jnp.exp(-o))
    out_ref[0] = jnp.where(lanei < 3, sig, o)


NSRT = 16384
SR, SC = 128, 128


def _k2a_body(s_ref, ts_ref, ix_ref):
    s = s_ref[0]
    ir = lax.broadcasted_iota(jnp.int32, (SR, SC), 0)
    ic = lax.broadcasted_iota(jnp.int32, (SR, SC), 1)
    i = ir * SC + ic
    idx = i

    k = 2
    while k <= NSRT:
        j = k // 2
        while j >= 1:
            if j < SC:
                axis, d = 1, j
                bitclear = (ic & j) == 0
            else:
                axis, d = 0, j // SC
                bitclear = (ir & (j // SC)) == 0
            ps = jnp.where(bitclear, jnp.roll(s, -d, axis=axis),
                           jnp.roll(s, d, axis=axis))
            pi = jnp.where(bitclear, jnp.roll(idx, -d, axis=axis),
                           jnp.roll(idx, d, axis=axis))
            lt = (s > ps) | ((s == ps) & (idx < pi))
            asc = (i & k) == 0
            sel = (bitclear == asc) == lt
            s = jnp.where(sel, s, ps)
            idx = jnp.where(sel, idx, pi)
            j //= 2
        k *= 2
    ts_ref[0] = s[0:8, :].reshape(1, NSEL)
    ix_ref[0] = idx[0:8, :].reshape(1, NSEL)


def _k2b_body(ts_ref, ix_ref, g_ref, so_ref, fi_ref, tab_ref, m_ref):
    idx = ix_ref[0, 0:1, :]
    lane = lax.broadcasted_iota(jnp.int32, (1, NSEL), 1)

    a = idx % 3
    hw = idx // 3
    wc = (hw % W).astype(jnp.float32)
    hr = (hw // W).astype(jnp.float32)
    wh0 = jnp.where(a == 0, _WS_HALF[0],
                    jnp.where(a == 1, _WS_HALF[1], _WS_HALF[2]))
    hh0 = jnp.where(a == 0, _HS_HALF[0],
                    jnp.where(a == 1, _HS_HALF[1], _HS_HALF[2]))
    cx0 = wc * 4.0
    cy0 = hr * 4.0
    ax1 = cx0 - wh0
    ax2 = cx0 + wh0
    ay1 = cy0 - hh0
    ay2 = cy0 + hh0
    aw = ax2 - ax1
    ah = ay2 - ay1
    acx = ax1 + 0.5 * aw
    acy = ay1 + 0.5 * ah

    dx = g_ref[0, 0:1, :]
    dy = g_ref[0, 1:2, :]
    dw = jnp.minimum(g_ref[0, 2:3, :], BBOX_XFORM_CLIP)
    dh = jnp.minimum(g_ref[0, 3:4, :], BBOX_XFORM_CLIP)
    pcx = dx * aw + acx
    pcy = dy * ah + acy
    pw = jnp.exp(dw) * aw
    ph = jnp.exp(dh) * ah
    x1 = jnp.clip(pcx - 0.5 * pw, 0.0, IMG_W)
    y1 = jnp.clip(pcy - 0.5 * ph, 0.0, IMG_H)
    x2 = jnp.clip(pcx + 0.5 * pw, 0.0, IMG_W)
    y2 = jnp.clip(pcy + 0.5 * ph, 0.0, IMG_H)

    areas = (x2 - x1) * (y2 - y1)

    def _ext(vec, ohf):
        return jnp.sum(jnp.sum(ohf * vec, axis=1, keepdims=True), axis=0,
                       keepdims=True)

    x1t = jnp.transpose(x1)
    y1t = jnp.transpose(y1)
    x2t = jnp.transpose(x2)
    y2t = jnp.transpose(y2)
    for g in range(NSEL // 128):
        sl = slice(g * 128, (g + 1) * 128)
        xb1 = x1t[sl, :]
        yb1 = y1t[sl, :]
        xb2 = x2t[sl, :]
        yb2 = y2t[sl, :]
        ab = (xb2 - xb1) * (yb2 - yb1)
        wv = jnp.maximum(jnp.minimum(xb2, x2) - jnp.maximum(xb1, x1), 0.0)
        hv = jnp.maximum(jnp.minimum(yb2, y2) - jnp.maximum(yb1, y1), 0.0)
        inter = wv * hv
        iou = inter / (ab + areas - inter + 1e-9)
        m_ref[sl, :] = (iou > NMS_TH).astype(jnp.float32)

    lane8 = lax.broadcasted_iota(jnp.int32, (1, 8), 1)

    def nms_blk(nb, keep):
        base8 = pl.multiple_of(nb * 8, 8)
        base = nb * 8
        rows = m_ref[pl.ds(base8, 8), :]
        ltm = (lane < base).astype(jnp.float32)
        ext = jnp.max(rows * keep * ltm, axis=1, keepdims=True)
        sub = pltpu.roll(rows, (NSEL - base) % NSEL, axis=1)[:, 0:8]
        kblk = jnp.ones((1, 8), jnp.float32)
        for q in range(8):
            si = jnp.max(sub[q:q + 1, :] * kblk
                         * (lane8 < q).astype(jnp.float32),
                         axis=1, keepdims=True)
            ki = jnp.where((si > 0.0) | (ext[q:q + 1, :] > 0.0), 0.0, 1.0)
            kblk = jnp.where(lane8 == q, ki, kblk)
        kfull = jnp.pad(kblk, ((0, 0), (0, NSEL - 8)))
        kfull = pltpu.roll(kfull, base, axis=1)
        inblk = (lane >= base) & (lane < base + 8)
        return jnp.where(inblk, kfull, keep)

    keep = lax.fori_loop(0, PRE_NMS // 8, nms_blk,
                         jnp.ones((1, NSEL), jnp.float32), unroll=2)

    tsv = ts_ref[0, 0:1, :]
    fs0 = jnp.where(lane < PRE_NMS,
                    jnp.where(keep > 0.0, tsv, -1.0), NEGF)

    fr = fs0.reshape(8, 128)
    ir8 = lax.broadcasted_iota(jnp.int32, (8, 128), 0)
    ic8 = lax.broadcasted_iota(jnp.int32, (8, 128), 1)
    i8 = ir8 * 128 + ic8
    pos = i8
    k = 2
    while k <= NSEL:
        j = k // 2
        while j >= 1:
            if j < 128:
                axis, d = 1, j
                bitclear = (ic8 & j) == 0
            else:
                axis, d = 0, j // 128
                bitclear = (ir8 & (j // 128)) == 0
            ps = jnp.where(bitclear, jnp.roll(fr, -d, axis=axis),
                           jnp.roll(fr, d, axis=axis))
            pp = jnp.where(bitclear, jnp.roll(pos, -d, axis=axis),
                           jnp.roll(pos, d, axis=axis))
            lt = (fr > ps) | ((fr == ps) & (pos < pp))
            asc = (i8 & k) == 0
            sel = (bitclear == asc) == lt
            fr = jnp.where(sel, fr, ps)
            pos = jnp.where(sel, pos, pp)
            j //= 2
        k *= 2
    so_ref[0] = fr[0:1, :]
    fi_ref[0] = pos[0:1, :]
    tab_ref[0, :, 0:1] = x1t
    tab_ref[0, :, 1:2] = y1t
    tab_ref[0, :, 2:3] = x2t
    tab_ref[0, :, 3:4] = y2t
    tab_ref[0, :, 4:16] = jnp.zeros((NSEL, 12), jnp.float32)


_SC_CORES = 2
_SC_SUBCORES = 16


@functools.lru_cache(maxsize=None)
def _sc_gather(n_rows):
    nw = _SC_CORES * _SC_SUBCORES
    per_w = n_rows // nw
    mesh = plsc.VectorSubcoreMesh(core_axis_name="c", subcore_axis_name="s")

    @functools.partial(
        pl.kernel, mesh=mesh,
        out_type=jax.ShapeDtypeStruct((n_rows, 16), jnp.float32),
        scratch_types=[
            pltpu.VMEM((per_w,), jnp.int32),
            pltpu.VMEM((per_w, 16), jnp.float32),
            pltpu.SemaphoreType.DMA,
        ],
        compiler_params=pltpu.CompilerParams(use_tc_tiling_on_sc=False),
    )
    def gather_k(idx_hbm, table_hbm, out_hbm, idx_v, rows_v, sem):
        wid = lax.axis_index("s") * _SC_CORES + lax.axis_index("c")
        base = wid * per_w
        pltpu.sync_copy(idx_hbm.at[pl.ds(base, per_w)], idx_v)
        pltpu.async_copy(table_hbm.at[idx_v], rows_v, sem).wait()
        pltpu.sync_copy(rows_v, out_hbm.at[pl.ds(base, per_w)])

    return gather_k


@jax.jit
def kernel(features, conv_w, conv_b, cls_w, cls_b, bbox_w, bbox_b):
    B = features.shape[0]
    xp = jnp.pad(features, ((0, 0), (0, 0), (1, 1), (1, 1)))
    xfT = xp.reshape(B, 256, HP * WP).transpose(0, 2, 1)
    xfT = jnp.pad(xfT, ((0, 0), (0, NFLAT - HP * WP), (0, 0)))
    w9 = jnp.stack([conv_w[:, :, i, j] for i in range(3)
                    for j in range(3)])
    whead = jnp.concatenate(
        [cls_w[:, :, 0, 0], bbox_w[:, :, 0, 0],
         jnp.zeros((1, 256), jnp.float32)], axis=0).T
    bhead = jnp.concatenate(
        [cls_b, bbox_b, jnp.zeros((1,), jnp.float32)]).reshape(1, 16)
    cbr = conv_b.reshape(1, 256)

    out = pl.pallas_call(
        _k1_body,
        grid=(B,),
        in_specs=[
            pl.BlockSpec((1, NFLAT, 256), lambda b: (b, 0, 0)),
            pl.BlockSpec((9, 256, 256), lambda b: (0, 0, 0)),
            pl.BlockSpec((1, 256), lambda b: (0, 0)),
            pl.BlockSpec((256, 16), lambda b: (0, 0)),
            pl.BlockSpec((1, 16), lambda b: (0, 0)),
        ],
        out_specs=pl.BlockSpec((1, NPIX, 16), lambda b: (b, 0, 0)),
        out_shape=jax.ShapeDtypeStruct((B, NPIX, 16), jnp.float32),
    )(xfT, w9, cbr, whead, bhead)

    out3 = out.reshape(B, H, WP, 16)[:, :, :W, :]
    scores = out3[..., :3].reshape(B, NANC)
    reg = out3[..., 3:15].reshape(B, NANC, 4)

    scores_p = jnp.pad(scores, ((0, 0), (0, NSRT - NANC)),
                       constant_values=-1.0).reshape(B, SR, SC)

    ts, ix = pl.pallas_call(
        _k2a_body,
        grid=(B,),
        in_specs=[
            pl.BlockSpec((1, SR, SC), lambda b: (b, 0, 0)),
        ],
        out_specs=[
            pl.BlockSpec((1, 1, NSEL), lambda b: (b, 0, 0)),
            pl.BlockSpec((1, 1, NSEL), lambda b: (b, 0, 0)),
        ],
        out_shape=[
            jax.ShapeDtypeStruct((B, 1, NSEL), jnp.float32),
            jax.ShapeDtypeStruct((B, 1, NSEL), jnp.int32),
        ],
    )(scores_p)

    table = jnp.pad(reg, ((0, 0), (0, 0), (0, 12))).reshape(B * NANC, 16)
    idxg = (ix[:, 0, :] +
            (jnp.arange(B, dtype=jnp.int32) * NANC)[:, None]).reshape(-1)
    rows = _sc_gather(B * NSEL)(idxg, table)
    gath = rows.reshape(B, NSEL, 16)[:, :, 0:4].transpose(0, 2, 1)

    scoresp, fi, tab = pl.pallas_call(
        _k2b_body,
        grid=(B,),
        in_specs=[
            pl.BlockSpec((1, 1, NSEL), lambda b: (b, 0, 0)),
            pl.BlockSpec((1, 1, NSEL), lambda b: (b, 0, 0)),
            pl.BlockSpec((1, 4, NSEL), lambda b: (b, 0, 0)),
        ],
        out_specs=[
            pl.BlockSpec((1, 1, 128), lambda b: (b, 0, 0)),
            pl.BlockSpec((1, 1, 128), lambda b: (b, 0, 0)),
            pl.BlockSpec((1, NSEL, 16), lambda b: (b, 0, 0)),
        ],
        out_shape=[
            jax.ShapeDtypeStruct((B, 1, 128), jnp.float32),
            jax.ShapeDtypeStruct((B, 1, 128), jnp.int32),
            jax.ShapeDtypeStruct((B, NSEL, 16), jnp.float32),
        ],
        scratch_shapes=[pltpu.VMEM((NSEL, NSEL), jnp.float32)],
    )(ts, ix, gath)

    fig = (fi[:, 0, :] +
           (jnp.arange(B, dtype=jnp.int32) * NSEL)[:, None]).reshape(-1)
    rows2 = _sc_gather(B * 128)(fig, tab.reshape(B * NSEL, 16))
    boxes = rows2.reshape(B, 128, 16)[:, :POST_NMS, 0:4]
    out_scores = scoresp[:, 0, :POST_NMS]
    return boxes, out_scores


# speedup vs baseline: 1.0915x; 1.0154x over previous
import functools

import numpy as np
import jax
import jax.numpy as jnp
from jax import lax
from jax.experimental import pallas as pl
from jax.experimental.pallas import tpu as pltpu
from jax.experimental.pallas import tpu_sc as plsc

PRE_NMS = 1000
POST_NMS = 100
NMS_TH = 0.7
IMG_W = 320.0
IMG_H = 240.0
BBOX_XFORM_CLIP = float(np.log(1000.0 / 16.0))

H, W = 60, 80
A = 3
NANC = H * W * A
NPADS = 15360
HP, WP = H + 2, W + 2
NPIX = H * WP
NFLAT = 5088
NSEL = 1024
BIGI = np.int32(1 << 30)
NEGF = np.float32(-3.0e38)

_ratios = np.asarray((0.5, 1.0, 2.0), dtype=np.float64)
_hr = np.sqrt(_ratios)
_WS_HALF = (32.0 / _hr / 2.0).astype(np.float32)
_HS_HALF = (32.0 * _hr / 2.0).astype(np.float32)


def _k1_body(xf_ref, w9_ref, cb_ref, wh_ref, bh_ref, out_ref):
    acc = jnp.zeros((NPIX, 256), jnp.float32)
    for di in range(3):
        for dj in range(3):
            t = di * 3 + dj
            off = di * WP + dj
            acc = acc + lax.dot_general(
                xf_ref[0, pl.ds(off, NPIX), :], w9_ref[t],
                dimension_numbers=(((1,), (1,)), ((), ())),
                preferred_element_type=jnp.float32)
    tact = jnp.maximum(acc + cb_ref[...], 0.0)
    o = jnp.dot(tact, wh_ref[...], preferred_element_type=jnp.float32)
    o = o + bh_ref[...]
    lanei = lax.broadcasted_iota(jnp.int32, (NPIX, 16), 1)
    sig = 1.0 / (1.0 + jnp.exp(-o))
    out_ref[0] = jnp.where(lanei < 3, sig, o)


NSRT = 16384
SR, SC = 128, 128


def _k2a_body(s_ref, ts_ref, ix_ref):
    s = s_ref[0]
    ir = lax.broadcasted_iota(jnp.int32, (SR, SC), 0)
    ic = lax.broadcasted_iota(jnp.int32, (SR, SC), 1)
    i = ir * SC + ic
    idx = i

    k = 2
    while k <= NSRT:
        j = k // 2
        while j >= 1:
            if j < SC:
                axis, d = 1, j
                bitclear = (ic & j) == 0
            else:
                axis, d = 0, j // SC
                bitclear = (ir & (j // SC)) == 0
            ps = jnp.where(bitclear, jnp.roll(s, -d, axis=axis),
                           jnp.roll(s, d, axis=axis))
            pi = jnp.where(bitclear, jnp.roll(idx, -d, axis=axis),
                           jnp.roll(idx, d, axis=axis))
            lt = (s > ps) | ((s == ps) & (idx < pi))
            asc = (i & k) == 0
            sel = (bitclear == asc) == lt
            s = jnp.where(sel, s, ps)
            idx = jnp.where(sel, idx, pi)
            j //= 2
        k *= 2
    ts_ref[0] = s[0:8, :].reshape(1, NSEL)
    ix_ref[0] = idx[0:8, :].reshape(1, NSEL)


def _k2b_body(ts_ref, ix_ref, g_ref, so_ref, fi_ref, tab_ref, m_ref):
    idx = ix_ref[0, 0:1, :]
    lane = lax.broadcasted_iota(jnp.int32, (1, NSEL), 1)

    a = idx % 3
    hw = idx // 3
    wc = (hw % W).astype(jnp.float32)
    hr = (hw // W).astype(jnp.float32)
    wh0 = jnp.where(a == 0, _WS_HALF[0],
                    jnp.where(a == 1, _WS_HALF[1], _WS_HALF[2]))
    hh0 = jnp.where(a == 0, _HS_HALF[0],
                    jnp.where(a == 1, _HS_HALF[1], _HS_HALF[2]))
    cx0 = wc * 4.0
    cy0 = hr * 4.0
    ax1 = cx0 - wh0
    ax2 = cx0 + wh0
    ay1 = cy0 - hh0
    ay2 = cy0 + hh0
    aw = ax2 - ax1
    ah = ay2 - ay1
    acx = ax1 + 0.5 * aw
    acy = ay1 + 0.5 * ah

    dx = g_ref[0, 0:1, :]
    dy = g_ref[0, 1:2, :]
    dw = jnp.minimum(g_ref[0, 2:3, :], BBOX_XFORM_CLIP)
    dh = jnp.minimum(g_ref[0, 3:4, :], BBOX_XFORM_CLIP)
    pcx = dx * aw + acx
    pcy = dy * ah + acy
    pw = jnp.exp(dw) * aw
    ph = jnp.exp(dh) * ah
    x1 = jnp.clip(pcx - 0.5 * pw, 0.0, IMG_W)
    y1 = jnp.clip(pcy - 0.5 * ph, 0.0, IMG_H)
    x2 = jnp.clip(pcx + 0.5 * pw, 0.0, IMG_W)
    y2 = jnp.clip(pcy + 0.5 * ph, 0.0, IMG_H)

    areas = (x2 - x1) * (y2 - y1)

    def _ext(vec, ohf):
        return jnp.sum(jnp.sum(ohf * vec, axis=1, keepdims=True), axis=0,
                       keepdims=True)

    x1t = jnp.transpose(x1)
    y1t = jnp.transpose(y1)
    x2t = jnp.transpose(x2)
    y2t = jnp.transpose(y2)
    for g in range(NSEL // 128):
        sl = slice(g * 128, (g + 1) * 128)
        xb1 = x1t[sl, :]
        yb1 = y1t[sl, :]
        xb2 = x2t[sl, :]
        yb2 = y2t[sl, :]
        ab = (xb2 - xb1) * (yb2 - yb1)
        wv = jnp.maximum(jnp.minimum(xb2, x2) - jnp.maximum(xb1, x1), 0.0)
        hv = jnp.maximum(jnp.minimum(yb2, y2) - jnp.maximum(yb1, y1), 0.0)
        inter = wv * hv
        iou = inter / (ab + areas - inter + 1e-9)
        m_ref[sl, :] = (iou > NMS_TH).astype(jnp.float32)

    lane8 = lax.broadcasted_iota(jnp.int32, (1, 8), 1)

    def nms_blk(nb, keep):
        base8 = pl.multiple_of(nb * 8, 8)
        base = nb * 8
        rows = m_ref[pl.ds(base8, 8), :]
        ltm = (lane < base).astype(jnp.float32)
        ext = jnp.max(rows * keep * ltm, axis=1, keepdims=True)
        sub = pltpu.roll(rows, (NSEL - base) % NSEL, axis=1)[:, 0:8]
        kblk = jnp.ones((1, 8), jnp.float32)
        for q in range(8):
            si = jnp.max(sub[q:q + 1, :] * kblk
                         * (lane8 < q).astype(jnp.float32),
                         axis=1, keepdims=True)
            ki = jnp.where((si > 0.0) | (ext[q:q + 1, :] > 0.0), 0.0, 1.0)
            kblk = jnp.where(lane8 == q, ki, kblk)
        kfull = jnp.pad(kblk, ((0, 0), (0, NSEL - 8)))
        kfull = pltpu.roll(kfull, base, axis=1)
        inblk = (lane >= base) & (lane < base + 8)
        return jnp.where(inblk, kfull, keep)

    keep = lax.fori_loop(0, PRE_NMS // 8, nms_blk,
                         jnp.ones((1, NSEL), jnp.float32), unroll=5)

    tsv = ts_ref[0, 0:1, :]
    fs0 = jnp.where(lane < PRE_NMS,
                    jnp.where(keep > 0.0, tsv, -1.0), NEGF)

    fr = fs0.reshape(8, 128)
    ir8 = lax.broadcasted_iota(jnp.int32, (8, 128), 0)
    ic8 = lax.broadcasted_iota(jnp.int32, (8, 128), 1)
    i8 = ir8 * 128 + ic8
    pos = i8
    k = 2
    while k <= NSEL:
        j = k // 2
        while j >= 1:
            if j < 128:
                axis, d = 1, j
                bitclear = (ic8 & j) == 0
            else:
                axis, d = 0, j // 128
                bitclear = (ir8 & (j // 128)) == 0
            ps = jnp.where(bitclear, jnp.roll(fr, -d, axis=axis),
                           jnp.roll(fr, d, axis=axis))
            pp = jnp.where(bitclear, jnp.roll(pos, -d, axis=axis),
                           jnp.roll(pos, d, axis=axis))
            lt = (fr > ps) | ((fr == ps) & (pos < pp))
            asc = (i8 & k) == 0
            sel = (bitclear == asc) == lt
            fr = jnp.where(sel, fr, ps)
            pos = jnp.where(sel, pos, pp)
            j //= 2
        k *= 2
    so_ref[0] = fr[0:1, :]
    fi_ref[0] = pos[0:1, :]
    tab_ref[0, :, 0:1] = x1t
    tab_ref[0, :, 1:2] = y1t
    tab_ref[0, :, 2:3] = x2t
    tab_ref[0, :, 3:4] = y2t
    tab_ref[0, :, 4:16] = jnp.zeros((NSEL, 12), jnp.float32)


_SC_CORES = 2
_SC_SUBCORES = 16


@functools.lru_cache(maxsize=None)
def _sc_gather(n_rows):
    nw = _SC_CORES * _SC_SUBCORES
    per_w = n_rows // nw
    mesh = plsc.VectorSubcoreMesh(core_axis_name="c", subcore_axis_name="s")

    @functools.partial(
        pl.kernel, mesh=mesh,
        out_type=jax.ShapeDtypeStruct((n_rows, 16), jnp.float32),
        scratch_types=[
            pltpu.VMEM((per_w,), jnp.int32),
            pltpu.VMEM((per_w, 16), jnp.float32),
            pltpu.SemaphoreType.DMA,
        ],
        compiler_params=pltpu.CompilerParams(use_tc_tiling_on_sc=False),
    )
    def gather_k(idx_hbm, table_hbm, out_hbm, idx_v, rows_v, sem):
        wid = lax.axis_index("s") * _SC_CORES + lax.axis_index("c")
        base = wid * per_w
        pltpu.sync_copy(idx_hbm.at[pl.ds(base, per_w)], idx_v)
        pltpu.async_copy(table_hbm.at[idx_v], rows_v, sem).wait()
        pltpu.sync_copy(rows_v, out_hbm.at[pl.ds(base, per_w)])

    return gather_k


@jax.jit
def kernel(features, conv_w, conv_b, cls_w, cls_b, bbox_w, bbox_b):
    B = features.shape[0]
    xp = jnp.pad(features, ((0, 0), (0, 0), (1, 1), (1, 1)))
    xfT = xp.reshape(B, 256, HP * WP).transpose(0, 2, 1)
    xfT = jnp.pad(xfT, ((0, 0), (0, NFLAT - HP * WP), (0, 0)))
    w9 = jnp.stack([conv_w[:, :, i, j] for i in range(3)
                    for j in range(3)])
    whead = jnp.concatenate(
        [cls_w[:, :, 0, 0], bbox_w[:, :, 0, 0],
         jnp.zeros((1, 256), jnp.float32)], axis=0).T
    bhead = jnp.concatenate(
        [cls_b, bbox_b, jnp.zeros((1,), jnp.float32)]).reshape(1, 16)
    cbr = conv_b.reshape(1, 256)

    out = pl.pallas_call(
        _k1_body,
        grid=(B,),
        in_specs=[
            pl.BlockSpec((1, NFLAT, 256), lambda b: (b, 0, 0)),
            pl.BlockSpec((9, 256, 256), lambda b: (0, 0, 0)),
            pl.BlockSpec((1, 256), lambda b: (0, 0)),
            pl.BlockSpec((256, 16), lambda b: (0, 0)),
            pl.BlockSpec((1, 16), lambda b: (0, 0)),
        ],
        out_specs=pl.BlockSpec((1, NPIX, 16), lambda b: (b, 0, 0)),
        out_shape=jax.ShapeDtypeStruct((B, NPIX, 16), jnp.float32),
    )(xfT, w9, cbr, whead, bhead)

    out3 = out.reshape(B, H, WP, 16)[:, :, :W, :]
    scores = out3[..., :3].reshape(B, NANC)
    reg = out3[..., 3:15].reshape(B, NANC, 4)

    scores_p = jnp.pad(scores, ((0, 0), (0, NSRT - NANC)),
                       constant_values=-1.0).reshape(B, SR, SC)

    ts, ix = pl.pallas_call(
        _k2a_body,
        grid=(B,),
        in_specs=[
            pl.BlockSpec((1, SR, SC), lambda b: (b, 0, 0)),
        ],
        out_specs=[
            pl.BlockSpec((1, 1, NSEL), lambda b: (b, 0, 0)),
            pl.BlockSpec((1, 1, NSEL), lambda b: (b, 0, 0)),
        ],
        out_shape=[
            jax.ShapeDtypeStruct((B, 1, NSEL), jnp.float32),
            jax.ShapeDtypeStruct((B, 1, NSEL), jnp.int32),
        ],
    )(scores_p)

    table = jnp.pad(reg, ((0, 0), (0, 0), (0, 12))).reshape(B * NANC, 16)
    idxg = (ix[:, 0, :] +
            (jnp.arange(B, dtype=jnp.int32) * NANC)[:, None]).reshape(-1)
    rows = _sc_gather(B * NSEL)(idxg, table)
    gath = rows.reshape(B, NSEL, 16)[:, :, 0:4].transpose(0, 2, 1)

    scoresp, fi, tab = pl.pallas_call(
        _k2b_body,
        grid=(B,),
        in_specs=[
            pl.BlockSpec((1, 1, NSEL), lambda b: (b, 0, 0)),
            pl.BlockSpec((1, 1, NSEL), lambda b: (b, 0, 0)),
            pl.BlockSpec((1, 4, NSEL), lambda b: (b, 0, 0)),
        ],
        out_specs=[
            pl.BlockSpec((1, 1, 128), lambda b: (b, 0, 0)),
            pl.BlockSpec((1, 1, 128), lambda b: (b, 0, 0)),
            pl.BlockSpec((1, NSEL, 16), lambda b: (b, 0, 0)),
        ],
        out_shape=[
            jax.ShapeDtypeStruct((B, 1, 128), jnp.float32),
            jax.ShapeDtypeStruct((B, 1, 128), jnp.int32),
            jax.ShapeDtypeStruct((B, NSEL, 16), jnp.float32),
        ],
        scratch_shapes=[pltpu.VMEM((NSEL, NSEL), jnp.float32)],
    )(ts, ix, gath)

    fig = (fi[:, 0, :] +
           (jnp.arange(B, dtype=jnp.int32) * NSEL)[:, None]).reshape(-1)
    rows2 = _sc_gather(B * 128)(fig, tab.reshape(B * NSEL, 16))
    boxes = rows2.reshape(B, 128, 16)[:, :POST_NMS, 0:4]
    out_scores = scoresp[:, 0, :POST_NMS]
    return boxes, out_scores


# speedup vs baseline: 1.1013x; 1.0090x over previous
import functools

import numpy as np
import jax
import jax.numpy as jnp
from jax import lax
from jax.experimental import pallas as pl
from jax.experimental.pallas import tpu as pltpu
from jax.experimental.pallas import tpu_sc as plsc

PRE_NMS = 1000
POST_NMS = 100
NMS_TH = 0.7
IMG_W = 320.0
IMG_H = 240.0
BBOX_XFORM_CLIP = float(np.log(1000.0 / 16.0))

H, W = 60, 80
A = 3
NANC = H * W * A
NPADS = 15360
HP, WP = H + 2, W + 2
NPIX = H * WP
NFLAT = 5088
NSEL = 1024
BIGI = np.int32(1 << 30)
NEGF = np.float32(-3.0e38)

_ratios = np.asarray((0.5, 1.0, 2.0), dtype=np.float64)
_hr = np.sqrt(_ratios)
_WS_HALF = (32.0 / _hr / 2.0).astype(np.float32)
_HS_HALF = (32.0 * _hr / 2.0).astype(np.float32)


def _k1_body(xf_ref, w9_ref, cb_ref, wh_ref, bh_ref, out_ref):
    acc = jnp.zeros((NPIX, 256), jnp.float32)
    for di in range(3):
        for dj in range(3):
            t = di * 3 + dj
            off = di * WP + dj
            acc = acc + lax.dot_general(
                xf_ref[0, pl.ds(off, NPIX), :], w9_ref[t],
                dimension_numbers=(((1,), (1,)), ((), ())),
                preferred_element_type=jnp.float32)
    tact = jnp.maximum(acc + cb_ref[...], 0.0)
    o = jnp.dot(tact, wh_ref[...], preferred_element_type=jnp.float32)
    o = o + bh_ref[...]
    lanei = lax.broadcasted_iota(jnp.int32, (NPIX, 16), 1)
    sig = 1.0 / (1.0 + jnp.exp(-o))
    out_ref[0] = jnp.where(lanei < 3, sig, o)


NSRT = 16384
SR, SC = 128, 128


def _k2a_body(s_ref, ts_ref, ix_ref):
    s = s_ref[0]
    ir = lax.broadcasted_iota(jnp.int32, (SR, SC), 0)
    ic = lax.broadcasted_iota(jnp.int32, (SR, SC), 1)
    i = ir * SC + ic
    idx = i

    k = 2
    while k <= NSRT:
        j = k // 2
        while j >= 1:
            if j < SC:
                axis, d = 1, j
                bitclear = (ic & j) == 0
            else:
                axis, d = 0, j // SC
                bitclear = (ir & (j // SC)) == 0
            ps = jnp.where(bitclear, jnp.roll(s, -d, axis=axis),
                           jnp.roll(s, d, axis=axis))
            pi = jnp.where(bitclear, jnp.roll(idx, -d, axis=axis),
                           jnp.roll(idx, d, axis=axis))
            lt = (s > ps) | ((s == ps) & (idx < pi))
            asc = (i & k) == 0
            sel = (bitclear == asc) == lt
            s = jnp.where(sel, s, ps)
            idx = jnp.where(sel, idx, pi)
            j //= 2
        k *= 2
    ts_ref[0] = s[0:8, :].reshape(1, NSEL)
    ix_ref[0] = idx[0:8, :].reshape(1, NSEL)


def _k2b_body(ts_ref, ix_ref, g_ref, so_ref, fi_ref, tab_ref, m_ref):
    idx = ix_ref[0, 0:1, :]
    lane = lax.broadcasted_iota(jnp.int32, (1, NSEL), 1)

    a = idx % 3
    hw = idx // 3
    wc = (hw % W).astype(jnp.float32)
    hr = (hw // W).astype(jnp.float32)
    wh0 = jnp.where(a == 0, _WS_HALF[0],
                    jnp.where(a == 1, _WS_HALF[1], _WS_HALF[2]))
    hh0 = jnp.where(a == 0, _HS_HALF[0],
                    jnp.where(a == 1, _HS_HALF[1], _HS_HALF[2]))
    cx0 = wc * 4.0
    cy0 = hr * 4.0
    ax1 = cx0 - wh0
    ax2 = cx0 + wh0
    ay1 = cy0 - hh0
    ay2 = cy0 + hh0
    aw = ax2 - ax1
    ah = ay2 - ay1
    acx = ax1 + 0.5 * aw
    acy = ay1 + 0.5 * ah

    dx = g_ref[0, 0:1, :]
    dy = g_ref[0, 1:2, :]
    dw = jnp.minimum(g_ref[0, 2:3, :], BBOX_XFORM_CLIP)
    dh = jnp.minimum(g_ref[0, 3:4, :], BBOX_XFORM_CLIP)
    pcx = dx * aw + acx
    pcy = dy * ah + acy
    pw = jnp.exp(dw) * aw
    ph = jnp.exp(dh) * ah
    x1 = jnp.clip(pcx - 0.5 * pw, 0.0, IMG_W)
    y1 = jnp.clip(pcy - 0.5 * ph, 0.0, IMG_H)
    x2 = jnp.clip(pcx + 0.5 * pw, 0.0, IMG_W)
    y2 = jnp.clip(pcy + 0.5 * ph, 0.0, IMG_H)

    areas = (x2 - x1) * (y2 - y1)

    def _ext(vec, ohf):
        return jnp.sum(jnp.sum(ohf * vec, axis=1, keepdims=True), axis=0,
                       keepdims=True)

    x1t = jnp.transpose(x1)
    y1t = jnp.transpose(y1)
    x2t = jnp.transpose(x2)
    y2t = jnp.transpose(y2)
    for g in range(NSEL // 128):
        sl = slice(g * 128, (g + 1) * 128)
        xb1 = x1t[sl, :]
        yb1 = y1t[sl, :]
        xb2 = x2t[sl, :]
        yb2 = y2t[sl, :]
        ab = (xb2 - xb1) * (yb2 - yb1)
        wv = jnp.maximum(jnp.minimum(xb2, x2) - jnp.maximum(xb1, x1), 0.0)
        hv = jnp.maximum(jnp.minimum(yb2, y2) - jnp.maximum(yb1, y1), 0.0)
        inter = wv * hv
        iou = inter / (ab + areas - inter + 1e-9)
        m_ref[sl, :] = (iou > NMS_TH).astype(jnp.float32)

    lane8 = lax.broadcasted_iota(jnp.int32, (1, 8), 1)

    def nms_blk(nb, keep):
        base8 = pl.multiple_of(nb * 8, 8)
        base = nb * 8
        rows = m_ref[pl.ds(base8, 8), :]
        ltm = (lane < base).astype(jnp.float32)
        ext = jnp.max(rows * keep * ltm, axis=1, keepdims=True)
        sub = pltpu.roll(rows, (NSEL - base) % NSEL, axis=1)[:, 0:8]
        kblk = jnp.ones((1, 8), jnp.float32)
        for q in range(8):
            si = jnp.max(sub[q:q + 1, :] * kblk
                         * (lane8 < q).astype(jnp.float32),
                         axis=1, keepdims=True)
            ki = jnp.where((si > 0.0) | (ext[q:q + 1, :] > 0.0), 0.0, 1.0)
            kblk = jnp.where(lane8 == q, ki, kblk)
        kfull = jnp.pad(kblk, ((0, 0), (0, NSEL - 8)))
        kfull = pltpu.roll(kfull, base, axis=1)
        inblk = (lane >= base) & (lane < base + 8)
        return jnp.where(inblk, kfull, keep)

    keep = lax.fori_loop(0, PRE_NMS // 8, nms_blk,
                         jnp.ones((1, NSEL), jnp.float32), unroll=25)

    tsv = ts_ref[0, 0:1, :]
    fs0 = jnp.where(lane < PRE_NMS,
                    jnp.where(keep > 0.0, tsv, -1.0), NEGF)

    fr = fs0.reshape(8, 128)
    ir8 = lax.broadcasted_iota(jnp.int32, (8, 128), 0)
    ic8 = lax.broadcasted_iota(jnp.int32, (8, 128), 1)
    i8 = ir8 * 128 + ic8
    pos = i8
    k = 2
    while k <= NSEL:
        j = k // 2
        while j >= 1:
            if j < 128:
                axis, d = 1, j
                bitclear = (ic8 & j) == 0
            else:
                axis, d = 0, j // 128
                bitclear = (ir8 & (j // 128)) == 0
            ps = jnp.where(bitclear, jnp.roll(fr, -d, axis=axis),
                           jnp.roll(fr, d, axis=axis))
            pp = jnp.where(bitclear, jnp.roll(pos, -d, axis=axis),
                           jnp.roll(pos, d, axis=axis))
            lt = (fr > ps) | ((fr == ps) & (pos < pp))
            asc = (i8 & k) == 0
            sel = (bitclear == asc) == lt
            fr = jnp.where(sel, fr, ps)
            pos = jnp.where(sel, pos, pp)
            j //= 2
        k *= 2
    so_ref[0] = fr[0:1, :]
    fi_ref[0] = pos[0:1, :]
    tab_ref[0, :, 0:1] = x1t
    tab_ref[0, :, 1:2] = y1t
    tab_ref[0, :, 2:3] = x2t
    tab_ref[0, :, 3:4] = y2t
    tab_ref[0, :, 4:16] = jnp.zeros((NSEL, 12), jnp.float32)


_SC_CORES = 2
_SC_SUBCORES = 16


@functools.lru_cache(maxsize=None)
def _sc_gather(n_rows):
    nw = _SC_CORES * _SC_SUBCORES
    per_w = n_rows // nw
    mesh = plsc.VectorSubcoreMesh(core_axis_name="c", subcore_axis_name="s")

    @functools.partial(
        pl.kernel, mesh=mesh,
        out_type=jax.ShapeDtypeStruct((n_rows, 16), jnp.float32),
        scratch_types=[
            pltpu.VMEM((per_w,), jnp.int32),
            pltpu.VMEM((per_w, 16), jnp.float32),
            pltpu.SemaphoreType.DMA,
        ],
        compiler_params=pltpu.CompilerParams(use_tc_tiling_on_sc=False),
    )
    def gather_k(idx_hbm, table_hbm, out_hbm, idx_v, rows_v, sem):
        wid = lax.axis_index("s") * _SC_CORES + lax.axis_index("c")
        base = wid * per_w
        pltpu.sync_copy(idx_hbm.at[pl.ds(base, per_w)], idx_v)
        pltpu.async_copy(table_hbm.at[idx_v], rows_v, sem).wait()
        pltpu.sync_copy(rows_v, out_hbm.at[pl.ds(base, per_w)])

    return gather_k


@jax.jit
def kernel(features, conv_w, conv_b, cls_w, cls_b, bbox_w, bbox_b):
    B = features.shape[0]
    xp = jnp.pad(features, ((0, 0), (0, 0), (1, 1), (1, 1)))
    xfT = xp.reshape(B, 256, HP * WP).transpose(0, 2, 1)
    xfT = jnp.pad(xfT, ((0, 0), (0, NFLAT - HP * WP), (0, 0)))
    w9 = jnp.stack([conv_w[:, :, i, j] for i in range(3)
                    for j in range(3)])
    whead = jnp.concatenate(
        [cls_w[:, :, 0, 0], bbox_w[:, :, 0, 0],
         jnp.zeros((1, 256), jnp.float32)], axis=0).T
    bhead = jnp.concatenate(
        [cls_b, bbox_b, jnp.zeros((1,), jnp.float32)]).reshape(1, 16)
    cbr = conv_b.reshape(1, 256)

    out = pl.pallas_call(
        _k1_body,
        grid=(B,),
        in_specs=[
            pl.BlockSpec((1, NFLAT, 256), lambda b: (b, 0, 0)),
            pl.BlockSpec((9, 256, 256), lambda b: (0, 0, 0)),
            pl.BlockSpec((1, 256), lambda b: (0, 0)),
            pl.BlockSpec((256, 16), lambda b: (0, 0)),
            pl.BlockSpec((1, 16), lambda b: (0, 0)),
        ],
        out_specs=pl.BlockSpec((1, NPIX, 16), lambda b: (b, 0, 0)),
        out_shape=jax.ShapeDtypeStruct((B, NPIX, 16), jnp.float32),
    )(xfT, w9, cbr, whead, bhead)

    out3 = out.reshape(B, H, WP, 16)[:, :, :W, :]
    scores = out3[..., :3].reshape(B, NANC)
    reg = out3[..., 3:15].reshape(B, NANC, 4)

    scores_p = jnp.pad(scores, ((0, 0), (0, NSRT - NANC)),
                       constant_values=-1.0).reshape(B, SR, SC)

    ts, ix = pl.pallas_call(
        _k2a_body,
        grid=(B,),
        in_specs=[
            pl.BlockSpec((1, SR, SC), lambda b: (b, 0, 0)),
        ],
        out_specs=[
            pl.BlockSpec((1, 1, NSEL), lambda b: (b, 0, 0)),
            pl.BlockSpec((1, 1, NSEL), lambda b: (b, 0, 0)),
        ],
        out_shape=[
            jax.ShapeDtypeStruct((B, 1, NSEL), jnp.float32),
            jax.ShapeDtypeStruct((B, 1, NSEL), jnp.int32),
        ],
    )(scores_p)

    table = jnp.pad(reg, ((0, 0), (0, 0), (0, 12))).reshape(B * NANC, 16)
    idxg = (ix[:, 0, :] +
            (jnp.arange(B, dtype=jnp.int32) * NANC)[:, None]).reshape(-1)
    rows = _sc_gather(B * NSEL)(idxg, table)
    gath = rows.reshape(B, NSEL, 16)[:, :, 0:4].transpose(0, 2, 1)

    scoresp, fi, tab = pl.pallas_call(
        _k2b_body,
        grid=(B,),
        in_specs=[
            pl.BlockSpec((1, 1, NSEL), lambda b: (b, 0, 0)),
            pl.BlockSpec((1, 1, NSEL), lambda b: (b, 0, 0)),
            pl.BlockSpec((1, 4, NSEL), lambda b: (b, 0, 0)),
        ],
        out_specs=[
            pl.BlockSpec((1, 1, 128), lambda b: (b, 0, 0)),
            pl.BlockSpec((1, 1, 128), lambda b: (b, 0, 0)),
            pl.BlockSpec((1, NSEL, 16), lambda b: (b, 0, 0)),
        ],
        out_shape=[
            jax.ShapeDtypeStruct((B, 1, 128), jnp.float32),
            jax.ShapeDtypeStruct((B, 1, 128), jnp.int32),
            jax.ShapeDtypeStruct((B, NSEL, 16), jnp.float32),
        ],
        scratch_shapes=[pltpu.VMEM((NSEL, NSEL), jnp.float32)],
    )(ts, ix, gath)

    fig = (fi[:, 0, :] +
           (jnp.arange(B, dtype=jnp.int32) * NSEL)[:, None]).reshape(-1)
    rows2 = _sc_gather(B * 128)(fig, tab.reshape(B * NSEL, 16))
    boxes = rows2.reshape(B, 128, 16)[:, :POST_NMS, 0:4]
    out_scores = scoresp[:, 0, :POST_NMS]
    return boxes, out_scores
